# baseline (speedup 1.0000x reference)
import functools

import jax
import jax.numpy as jnp
from jax import lax
from jax.experimental import pallas as pl
from jax.experimental.pallas import tpu as pltpu
from jax.experimental.pallas import tpu_sc as plsc

B = 16384
VOCAB = 100000
HV = 51200
EMB = 64
EPS = 1e-3
NC, NS = 2, 16
NW = NC * NS
BPW = B // NW
CHUNK = 128
NCHUNK = BPW // CHUNK
NROW = B // CHUNK
TCBLK = 2048
NBLK = B // TCBLK
RPB = TCBLK // CHUNK
NBIN = 96
OB = 12800
TGRID = HV // OB


def _transpose_pack_kernel(embA_ref, embB_ref, gm_ref, grm_ref, cm_ref,
                           out_ref, code_ref):
    i = pl.program_id(0)
    f = lax.broadcasted_iota(jnp.int32, (EMB, 128), 0)
    l = lax.broadcasted_iota(jnp.int32, (EMB, 128), 1)
    identA = (f == l).astype(jnp.float32)
    identB = (f + EMB == l).astype(jnp.float32)
    out_ref[...] = (
        lax.dot_general(embA_ref[...], identA, (((0,), (0,)), ((), ())),
                        preferred_element_type=jnp.float32)
        + lax.dot_general(embB_ref[...], identB, (((0,), (0,)), ((), ())),
                          preferred_element_type=jnp.float32))
    a = pl.ds(i * OB, OB)
    b = pl.ds(HV + i * OB, OB)
    lo = gm_ref[a] | (grm_ref[a] << 5) | (cm_ref[a] << 10)
    hi = gm_ref[b] | (grm_ref[b] << 5) | (cm_ref[b] << 10)
    code_ref[a] = lo | (hi << 15)


def _transpose_pack(embT, group_map, graph_map, colour_map):
    vfull = pl.BlockSpec((2 * HV,), lambda i: (0,))
    return pl.pallas_call(
        _transpose_pack_kernel,
        grid=(TGRID,),
        in_specs=[pl.BlockSpec((EMB, OB), lambda i: (0, i)),
                  pl.BlockSpec((EMB, OB), lambda i: (0, i + TGRID)),
                  vfull, vfull, vfull],
        out_specs=[pl.BlockSpec((OB, 128), lambda i: (i, 0)),
                   pl.BlockSpec((HV,), lambda i: (0,))],
        out_shape=[jax.ShapeDtypeStruct((HV, 128), jnp.float32),
                   jax.ShapeDtypeStruct((HV,), jnp.int32)],
    )(embT, embT, group_map, graph_map, colour_map)


def _sc_gather(emb2, code, idx3):
    mesh = plsc.VectorSubcoreMesh(core_axis_name="c", subcore_axis_name="s")

    @functools.partial(
        pl.kernel,
        mesh=mesh,
        out_type=(
            jax.ShapeDtypeStruct((B, 128), jnp.float32),
            jax.ShapeDtypeStruct((NROW, CHUNK), jnp.int32),
        ),
        scratch_types=[
            pltpu.VMEM((NCHUNK, CHUNK), jnp.int32),
            pltpu.VMEM((BPW, 128), jnp.float32),
            pltpu.VMEM((NCHUNK, CHUNK), jnp.int32),
            pltpu.SemaphoreType.DMA,
        ],
    )
    def k(emb_hbm, code_hbm, idx_hbm, x_out, cats_out, idx_v, rows_v, val_v, sem):
        wid = lax.axis_index("s") * NC + lax.axis_index("c")
        pltpu.sync_copy(idx_hbm.at[wid], idx_v)
        copies = []
        for j in range(NCHUNK):
            copies.append(pltpu.async_copy(
                emb_hbm.at[idx_v.at[j]], rows_v.at[pl.ds(j * CHUNK, CHUNK)], sem))
            copies.append(pltpu.async_copy(
                code_hbm.at[idx_v.at[j]], val_v.at[j], sem))
        for c in copies:
            c.wait()
        pltpu.sync_copy(rows_v, x_out.at[pl.ds(wid * BPW, BPW)])
        pltpu.sync_copy(val_v, cats_out.at[pl.ds(wid * NCHUNK, NCHUNK)])

    return k(emb2, code, idx3)


def _onehot_t(crows):
    u = lax.broadcasted_iota(jnp.int32, (NBIN, 1, 1), 0)
    shift = jnp.where(u < 32, 0, jnp.where(u < 64, 5, 10))
    binval = u % 32
    val = lax.shift_right_logical(crows[None, :, :], shift) & 31
    ohf = jnp.where(val == binval, jnp.float32(1), jnp.float32(0))
    return ohf.astype(jnp.bfloat16)


def _fdot(a, b):
    return jnp.dot(a, b, preferred_element_type=jnp.float32)


def _tdot(a, b):
    return lax.dot_general(a, b, (((0,), (0,)), ((), ())),
                           preferred_element_type=jnp.float32)


def _select_block(x_ref, pair_rows, flag_rows, blk):
    codes = lax.shift_right_logical(pair_rows, flag_rows * 15) & 0x7FFF
    flagf = flag_rows.astype(jnp.float32)
    fcol = jnp.concatenate(
        [jnp.transpose(flagf[c:c + 1, :]) for c in range(RPB)], axis=0)
    xb = x_ref[pl.ds(blk * TCBLK, TCBLK), :]
    xsel = jnp.where(fcol > 0, xb[:, EMB:128], xb[:, 0:EMB])
    return xsel.astype(jnp.bfloat16), codes


def _tc_kernel(x_ref, cats_ref, aid_ref, gtT_ref, grtT_ref, ctT_ref, w_ref,
               gamma_ref, beta_ref,
               out_ref, gw_ref, bias_ref, w1b_ref):
    i = pl.program_id(0)
    binv = jnp.float32(1.0 / B)

    @pl.when(i == 0)
    def _():
        onesb = jnp.ones((1, TCBLK), jnp.bfloat16)
        acc_s = jnp.zeros((1, EMB), jnp.float32)
        acc_q = jnp.zeros((1, EMB), jnp.float32)
        ohacc = jnp.zeros((NBIN, CHUNK), jnp.float32)
        for k in range(NBLK):
            pair = cats_ref[k * RPB:(k + 1) * RPB, :]
            flag = (aid_ref[k * RPB:(k + 1) * RPB, :] >= HV).astype(jnp.int32)
            xc, codes = _select_block(x_ref, pair, flag, k)
            acc_s += _fdot(onesb, xc)
            acc_q += _fdot(onesb, xc * xc)
            oh3 = _onehot_t(codes)
            for c in range(RPB):
                ohacc += oh3[:, c, :].astype(jnp.float32)
        cnt_col = jnp.sum(ohacc, axis=1, keepdims=True)
        g1 = gamma_ref[0:EMB].reshape(1, EMB)
        b1 = beta_ref[0:EMB].reshape(1, EMB)
        mean1 = acc_s * binv
        var1 = acc_q * binv - mean1 * mean1
        s1 = g1 * lax.rsqrt(var1 + EPS)
        w1 = w_ref[0:EMB, :]
        w1b_ref[...] = (w1 * jnp.transpose(s1)).astype(jnp.bfloat16)
        gw_ref[...] = jnp.zeros((NBIN, 128), jnp.bfloat16)
        bias = _fdot(b1 - mean1 * s1, w1)
        for off, tT_ref, nc, wlo, whi in (
                (0, gtT_ref, 20, 64, 74),
                (32, grtT_ref, 31, 74, 89),
                (64, ctT_ref, 21, 89, 99)):
            tT = tT_ref[...]
            wp = w_ref[wlo:whi, :]
            nf = whi - wlo
            g = jnp.transpose(gamma_ref[wlo:whi].reshape(1, nf))
            b = jnp.transpose(beta_ref[wlo:whi].reshape(1, nf))
            cnt = cnt_col[off:off + nc, :]
            mean = _fdot(tT, cnt) * binv
            ex2 = _fdot(tT * tT, cnt) * binv
            var = ex2 - mean * mean
            s = g * lax.rsqrt(var + EPS)
            gw_ref[off:off + nc, :] = _tdot(tT * s, wp).astype(jnp.bfloat16)
            bias += _tdot(b - mean * s, wp)
        bias_ref[...] = bias

    pair = cats_ref[pl.ds(i * RPB, RPB), :]
    flag = (aid_ref[pl.ds(i * RPB, RPB), :] >= HV).astype(jnp.int32)
    x, codes = _select_block(x_ref, pair, flag, i)
    base = _fdot(x, w1b_ref[...]) + bias_ref[...]
    oh3 = _onehot_t(codes)
    gw = gw_ref[...]
    for c in range(RPB):
        out_ref[c * CHUNK:(c + 1) * CHUNK, :] = (
            base[c * CHUNK:(c + 1) * CHUNK, :] + _tdot(oh3[:, c, :], gw))


def _tc_fuse(x, cats2, aid2, gtT, grtT, ctT, W, gamma, beta):
    full = lambda shape: pl.BlockSpec(shape, lambda i: tuple(0 for _ in shape))
    return pl.pallas_call(
        _tc_kernel,
        grid=(NBLK,),
        in_specs=[full((B, 128)), full((NROW, CHUNK)), full((NROW, CHUNK)),
                  full((10, 20)), full((15, 31)), full((10, 21)),
                  full((99, 128)), full((99,)), full((99,))],
        out_specs=pl.BlockSpec((TCBLK, 128), lambda i: (i, 0)),
        out_shape=jax.ShapeDtypeStruct((B, 128), jnp.float32),
        scratch_shapes=[
            pltpu.VMEM((NBIN, 128), jnp.bfloat16),
            pltpu.VMEM((1, 128), jnp.float32),
            pltpu.VMEM((EMB, 128), jnp.bfloat16),
        ],
    )(x, cats2, aid2, gtT, grtT, ctT, W, gamma, beta)


def kernel(article_id, group_map, graph_map, colour_map,
           emb_table, group_table, graph_table, colour_table,
           gamma, beta, W):
    pad = lambda m: jnp.pad(m, (0, 2 * HV - VOCAB))
    emb2, code = _transpose_pack(emb_table.T, pad(group_map),
                                 pad(graph_map), pad(colour_map))
    idx3 = (article_id % HV).reshape(NW, NCHUNK, CHUNK)
    aid2 = article_id.reshape(NROW, CHUNK)

    x, cats2 = _sc_gather(emb2, code, idx3)

    return _tc_fuse(x, cats2, aid2, group_table.T, graph_table.T,
                    colour_table.T, W, gamma, beta)


# speedup vs baseline: 1.1900x; 1.1900x over previous
import functools

import jax
import jax.numpy as jnp
from jax import lax
from jax.experimental import pallas as pl
from jax.experimental.pallas import tpu as pltpu
from jax.experimental.pallas import tpu_sc as plsc

B = 16384
VOCAB = 100000
EMB = 64
EPS = 1e-3
NC, NS = 2, 16
NW = NC * NS
BPW = B // NW
CHUNK = 128
NCHUNK = BPW // CHUNK
NROW = B // CHUNK
TCBLK = 2048
NBLK = B // TCBLK
RPB = TCBLK // CHUNK
NBIN = 96
TBLK = 16384
TGRID = (VOCAB + TBLK - 1) // TBLK


def _transpose_pad_kernel(embT_ref, gm_ref, grm_ref, cm_ref, out_ref, code_ref):
    ident = (lax.broadcasted_iota(jnp.int32, (EMB, 128), 0)
             == lax.broadcasted_iota(jnp.int32, (EMB, 128), 1)
             ).astype(jnp.float32)
    out_ref[...] = lax.dot_general(
        embT_ref[...], ident, (((0,), (0,)), ((), ())),
        preferred_element_type=jnp.float32)
    code_ref[...] = (gm_ref[...] | (grm_ref[...] << 5) | (cm_ref[...] << 10))


def _transpose_pad(embT, group_map, graph_map, colour_map):
    return pl.pallas_call(
        _transpose_pad_kernel,
        grid=(TGRID,),
        in_specs=[pl.BlockSpec((EMB, TBLK), lambda i: (0, i)),
                  pl.BlockSpec((TBLK,), lambda i: (i,)),
                  pl.BlockSpec((TBLK,), lambda i: (i,)),
                  pl.BlockSpec((TBLK,), lambda i: (i,))],
        out_specs=[pl.BlockSpec((TBLK, 128), lambda i: (i, 0)),
                   pl.BlockSpec((TBLK,), lambda i: (i,))],
        out_shape=[jax.ShapeDtypeStruct((VOCAB, 128), jnp.float32),
                   jax.ShapeDtypeStruct((VOCAB,), jnp.int32)],
    )(embT, group_map, graph_map, colour_map)


def _sc_gather(emb128, code, idx3):
    mesh = plsc.VectorSubcoreMesh(core_axis_name="c", subcore_axis_name="s")

    @functools.partial(
        pl.kernel,
        mesh=mesh,
        out_type=(
            jax.ShapeDtypeStruct((B, 128), jnp.float32),
            jax.ShapeDtypeStruct((NROW, CHUNK), jnp.int32),
        ),
        scratch_types=[
            pltpu.VMEM((NCHUNK, CHUNK), jnp.int32),
            pltpu.VMEM((BPW, 128), jnp.float32),
            pltpu.VMEM((NCHUNK, CHUNK), jnp.int32),
            pltpu.SemaphoreType.DMA,
        ],
    )
    def k(emb_hbm, code_hbm, idx_hbm, x_out, cats_out, idx_v, rows_v, val_v, sem):
        wid = lax.axis_index("s") * NC + lax.axis_index("c")
        pltpu.sync_copy(idx_hbm.at[wid], idx_v)
        copies = []
        for j in range(NCHUNK):
            copies.append(pltpu.async_copy(
                emb_hbm.at[idx_v.at[j]], rows_v.at[pl.ds(j * CHUNK, CHUNK)], sem))
            copies.append(pltpu.async_copy(
                code_hbm.at[idx_v.at[j]], val_v.at[j], sem))
        for c in copies:
            c.wait()
        pltpu.sync_copy(rows_v, x_out.at[pl.ds(wid * BPW, BPW)])
        pltpu.sync_copy(val_v, cats_out.at[pl.ds(wid * NCHUNK, NCHUNK)])

    return k(emb128, code, idx3)


def _onehot_t(crows):
    u = lax.broadcasted_iota(jnp.int32, (NBIN, 1, 1), 0)
    shift = jnp.where(u < 32, 0, jnp.where(u < 64, 5, 10))
    binval = u % 32
    val = lax.shift_right_logical(crows[None, :, :], shift) & 31
    ohf = jnp.where(val == binval, jnp.float32(1), jnp.float32(0))
    return ohf.astype(jnp.bfloat16)


def _fdot(a, b):
    return jnp.dot(a, b, preferred_element_type=jnp.float32)


def _tdot(a, b):
    return lax.dot_general(a, b, (((0,), (0,)), ((), ())),
                           preferred_element_type=jnp.float32)


def _tc_kernel(x_ref, cats_ref, gtT_ref, grtT_ref, ctT_ref, w_ref,
               gamma_ref, beta_ref,
               out_ref, gw_ref, bias_ref, w1b_ref):
    i = pl.program_id(0)
    binv = jnp.float32(1.0 / B)

    @pl.when(i == 0)
    def _():
        onesb = jnp.ones((1, TCBLK), jnp.bfloat16)
        acc_s = jnp.zeros((1, 128), jnp.float32)
        acc_q = jnp.zeros((1, 128), jnp.float32)
        ohacc = jnp.zeros((NBIN, CHUNK), jnp.float32)
        for k in range(NBLK):
            xc = x_ref[k * TCBLK:(k + 1) * TCBLK, :].astype(jnp.bfloat16)
            acc_s += _fdot(onesb, xc)
            acc_q += _fdot(onesb, xc * xc)
            oh3 = _onehot_t(cats_ref[k * RPB:(k + 1) * RPB, :])
            for c in range(RPB):
                ohacc += oh3[:, c, :].astype(jnp.float32)
        cnt_col = jnp.sum(ohacc, axis=1, keepdims=True)
        g1 = gamma_ref[0:EMB].reshape(1, EMB)
        b1 = beta_ref[0:EMB].reshape(1, EMB)
        mean1 = acc_s[:, 0:EMB] * binv
        var1 = acc_q[:, 0:EMB] * binv - mean1 * mean1
        s1 = g1 * lax.rsqrt(var1 + EPS)
        w1 = w_ref[0:EMB, :]
        w1b_ref[0:EMB, :] = (w1 * jnp.transpose(s1)).astype(jnp.bfloat16)
        w1b_ref[EMB:128, :] = jnp.zeros((128 - EMB, 128), jnp.bfloat16)
        gw_ref[...] = jnp.zeros((NBIN, 128), jnp.bfloat16)
        bias = _fdot(b1 - mean1 * s1, w1)
        for off, tT_ref, nc, wlo, whi in (
                (0, gtT_ref, 20, 64, 74),
                (32, grtT_ref, 31, 74, 89),
                (64, ctT_ref, 21, 89, 99)):
            tT = tT_ref[...]
            wp = w_ref[wlo:whi, :]
            nf = whi - wlo
            g = jnp.transpose(gamma_ref[wlo:whi].reshape(1, nf))
            b = jnp.transpose(beta_ref[wlo:whi].reshape(1, nf))
            cnt = cnt_col[off:off + nc, :]
            mean = _fdot(tT, cnt) * binv
            ex2 = _fdot(tT * tT, cnt) * binv
            var = ex2 - mean * mean
            s = g * lax.rsqrt(var + EPS)
            gw_ref[off:off + nc, :] = _tdot(tT * s, wp).astype(jnp.bfloat16)
            bias += _tdot(b - mean * s, wp)
        bias_ref[...] = bias

    x = x_ref[pl.ds(i * TCBLK, TCBLK), :].astype(jnp.bfloat16)
    base = _fdot(x, w1b_ref[...]) + bias_ref[...]
    oh3 = _onehot_t(cats_ref[pl.ds(i * RPB, RPB), :])
    gw = gw_ref[...]
    for c in range(RPB):
        out_ref[c * CHUNK:(c + 1) * CHUNK, :] = (
            base[c * CHUNK:(c + 1) * CHUNK, :] + _tdot(oh3[:, c, :], gw))


def _tc_fuse(x, cats2, gtT, grtT, ctT, W, gamma, beta):
    full = lambda shape: pl.BlockSpec(shape, lambda i: tuple(0 for _ in shape))
    return pl.pallas_call(
        _tc_kernel,
        grid=(NBLK,),
        in_specs=[full((B, 128)), full((NROW, CHUNK)),
                  full((10, 20)), full((15, 31)), full((10, 21)),
                  full((99, 128)), full((99,)), full((99,))],
        out_specs=pl.BlockSpec((TCBLK, 128), lambda i: (i, 0)),
        out_shape=jax.ShapeDtypeStruct((B, 128), jnp.float32),
        scratch_shapes=[
            pltpu.VMEM((NBIN, 128), jnp.bfloat16),
            pltpu.VMEM((1, 128), jnp.float32),
            pltpu.VMEM((128, 128), jnp.bfloat16),
        ],
    )(x, cats2, gtT, grtT, ctT, W, gamma, beta)


def kernel(article_id, group_map, graph_map, colour_map,
           emb_table, group_table, graph_table, colour_table,
           gamma, beta, W):
    emb128, code = _transpose_pad(emb_table.T, group_map, graph_map, colour_map)
    idx3 = article_id.reshape(NW, NCHUNK, CHUNK)

    x, cats2 = _sc_gather(emb128, code, idx3)

    return _tc_fuse(x, cats2, group_table.T, graph_table.T, colour_table.T,
                    W, gamma, beta)
